# Initial kernel scaffold; baseline (speedup 1.0000x reference)
import jax
import jax.numpy as jnp
from jax.experimental import pallas as pl


def kernel(x, edge_index, W1, b1, W2, b2, Wfc, bfc):
    raise NotImplementedError("write your pallas kernel here")


# speedup vs baseline: 4.1125x; 4.1125x over previous
import jax
import jax.numpy as jnp
from jax import lax
from jax.experimental import pallas as pl
from jax.experimental.pallas import tpu as pltpu
from jax.experimental.pallas import tpu_sc as plsc

N = 10000
NP = 10240
E = 320000
F = 128
H = 256
NCLS = 18
NC = 2
NS = 16
CH = 80
ROWS_T = NP // NS
C1 = E // (NC * NS * CH)
C2 = E // (NS * CH)
BR = NP // 16

import functools


@functools.cache
def _mesh():
    return plsc.VectorSubcoreMesh(core_axis_name="c", subcore_axis_name="s",
                                  num_cores=NC, num_subcores=NS)


def _deg_body(src3, dst3, oh2, zfeat, deg, idxs_v, idxd_v, ohs_v, ohd_v, acc_sh):
    c = lax.axis_index("c")
    s = lax.axis_index("s")
    r0 = s * ROWS_T
    pltpu.sync_copy(zfeat.at[pl.ds(r0, ROWS_T)], acc_sh.at[pl.ds(r0, ROWS_T)])
    pltpu.sync_copy(oh2.at[0], ohs_v)
    pltpu.sync_copy(oh2.at[1], ohd_v)
    plsc.subcore_barrier()

    def chunk(j, carry):
        pltpu.sync_copy(src3.at[c, s, j], idxs_v)
        pltpu.sync_copy(ohs_v, acc_sh.at[idxs_v], add=True)
        pltpu.sync_copy(dst3.at[c, s, j], idxd_v)
        pltpu.sync_copy(ohd_v, acc_sh.at[idxd_v], add=True)
        return carry

    lax.fori_loop(0, C1, chunk, 0)
    plsc.subcore_barrier()
    pltpu.sync_copy(acc_sh.at[pl.ds(r0, ROWS_T)], deg.at[c, pl.ds(r0, ROWS_T)])


@functools.cache
def _deg_call():
    return pl.kernel(
        _deg_body,
        out_type=jax.ShapeDtypeStruct((NC, NP, F), jnp.float32),
        mesh=_mesh(),
        scratch_types=[
            pltpu.VMEM((CH,), jnp.int32),
            pltpu.VMEM((CH,), jnp.int32),
            pltpu.VMEM((CH, F), jnp.float32),
            pltpu.VMEM((CH, F), jnp.float32),
            pltpu.VMEM_SHARED((NP, F), jnp.float32),
        ],
    )


def _make_spmm(nchunks, tbl_rows):
    def body(tbl, src3, dst3, zfeat, out, idxg_v, idxs_v, rows_v, acc_sh, sem):
        c = lax.axis_index("c")
        s = lax.axis_index("s")
        r0 = s * ROWS_T
        pltpu.sync_copy(zfeat.at[pl.ds(r0, ROWS_T)], acc_sh.at[pl.ds(r0, ROWS_T)])
        pltpu.sync_copy(src3.at[c, s], idxg_v)
        plsc.subcore_barrier()

        def chunk(j, carry):
            pltpu.sync_copy(dst3.at[c, s, j], idxs_v)
            pltpu.async_copy(tbl.at[idxg_v.at[j]], rows_v, sem).wait()
            pltpu.sync_copy(rows_v, acc_sh.at[idxs_v], add=True)
            return carry

        lax.fori_loop(0, nchunks, chunk, 0)
        plsc.subcore_barrier()
        pltpu.sync_copy(acc_sh.at[pl.ds(r0, ROWS_T)], out.at[c, pl.ds(r0, ROWS_T)])

    return pl.kernel(
        body,
        out_type=jax.ShapeDtypeStruct((NC, NP, F), jnp.float32),
        mesh=_mesh(),
        scratch_types=[
            pltpu.VMEM((nchunks, CH), jnp.int32),
            pltpu.VMEM((CH,), jnp.int32),
            pltpu.VMEM((CH, F), jnp.float32),
            pltpu.VMEM_SHARED((NP, F), jnp.float32),
            pltpu.SemaphoreType.DMA,
        ],
    )


_spmm1_call = functools.cache(lambda: _make_spmm(C1, NP))
_spmm2_call = functools.cache(lambda: _make_spmm(C2, NC * NP))


def _prep_body(deg_ref, x_ref, h0_ref, ns_ref, nd_ref):
    dsum = deg_ref[0] + deg_ref[1]
    ns = lax.rsqrt(jnp.maximum(dsum[:, 0:1], 1.0))
    nd = lax.rsqrt(jnp.maximum(dsum[:, 1:2], 1.0))
    h0_ref[...] = x_ref[...] * ns
    ns_ref[...] = ns
    nd_ref[...] = nd


def _prep_call(deg, x_p):
    return pl.pallas_call(
        _prep_body,
        grid=(NP // BR,),
        in_specs=[
            pl.BlockSpec((NC, BR, F), lambda i: (0, i, 0)),
            pl.BlockSpec((BR, F), lambda i: (i, 0)),
        ],
        out_specs=[
            pl.BlockSpec((BR, F), lambda i: (i, 0)),
            pl.BlockSpec((BR, 1), lambda i: (i, 0)),
            pl.BlockSpec((BR, 1), lambda i: (i, 0)),
        ],
        out_shape=[jax.ShapeDtypeStruct((NP, F), jnp.float32),
                   jax.ShapeDtypeStruct((NP, 1), jnp.float32),
                   jax.ShapeDtypeStruct((NP, 1), jnp.float32)],
    )(deg, x_p)


def _l1_body(a_ref, nd_ref, ns_ref, w_ref, b_ref, g_ref):
    agg = (a_ref[0] + a_ref[1]) * nd_ref[...]
    h = jnp.dot(agg, w_ref[...], preferred_element_type=jnp.float32,
                precision=lax.Precision.HIGHEST) + b_ref[...]
    h = jnp.maximum(h, 0.0) * ns_ref[...]
    g_ref[0] = h[:, :F]
    g_ref[1] = h[:, F:]


def _l1_call(agg1, nd, ns, W1, b1):
    return pl.pallas_call(
        _l1_body,
        grid=(NP // BR,),
        in_specs=[
            pl.BlockSpec((NC, BR, F), lambda i: (0, i, 0)),
            pl.BlockSpec((BR, 1), lambda i: (i, 0)),
            pl.BlockSpec((BR, 1), lambda i: (i, 0)),
            pl.BlockSpec((F, H), lambda i: (0, 0)),
            pl.BlockSpec((1, H), lambda i: (0, 0)),
        ],
        out_specs=pl.BlockSpec((NC, BR, F), lambda i: (0, i, 0)),
        out_shape=jax.ShapeDtypeStruct((NC, NP, F), jnp.float32),
    )(agg1, nd, ns, W1, b1)


def _l2_body(a_ref, nd_ref, w_ref, b_ref, wfc_ref, bfc_ref, o_ref, acc_ref):
    i = pl.program_id(0)

    @pl.when(i == 0)
    def _():
        acc_ref[...] = jnp.zeros_like(acc_ref)

    agg = jnp.concatenate([a_ref[0], a_ref[1]], axis=1) * nd_ref[...]
    h = jnp.dot(agg, w_ref[...], preferred_element_type=jnp.float32,
                precision=lax.Precision.HIGHEST) + b_ref[...]
    h = jnp.maximum(h, 0.0)
    rid = lax.broadcasted_iota(jnp.int32, h.shape, 0) + i * BR
    h = jnp.where(rid < N, h, 0.0)
    acc_ref[...] += jnp.sum(h, axis=0, keepdims=True)

    @pl.when(i == NP // BR - 1)
    def _():
        hg = acc_ref[...] * (1.0 / N)
        o_ref[...] = jnp.dot(hg, wfc_ref[...], preferred_element_type=jnp.float32,
                             precision=lax.Precision.HIGHEST) + bfc_ref[...]


def _l2_call(agg2, nd, W2, b2, wfc_p, bfc_p):
    return pl.pallas_call(
        _l2_body,
        grid=(NP // BR,),
        in_specs=[
            pl.BlockSpec((NC, BR, F), lambda i: (0, i, 0)),
            pl.BlockSpec((BR, 1), lambda i: (i, 0)),
            pl.BlockSpec((H, H), lambda i: (0, 0)),
            pl.BlockSpec((1, H), lambda i: (0, 0)),
            pl.BlockSpec((H, 128), lambda i: (0, 0)),
            pl.BlockSpec((1, 128), lambda i: (0, 0)),
        ],
        out_specs=pl.BlockSpec((1, 128), lambda i: (0, 0)),
        out_shape=jax.ShapeDtypeStruct((1, 128), jnp.float32),
        scratch_shapes=[pltpu.VMEM((1, H), jnp.float32)],
    )(agg2, nd, W2, b2, wfc_p, bfc_p)


def kernel(x, edge_index, W1, b1, W2, b2, Wfc, bfc):
    src = edge_index[0].astype(jnp.int32)
    dst = edge_index[1].astype(jnp.int32)
    x_p = jnp.zeros((NP, F), jnp.float32).at[:N].set(x)
    src1 = src.reshape(NC, NS, C1, CH)
    dst1 = dst.reshape(NC, NS, C1, CH)
    src2 = jnp.stack([src, src + NP]).reshape(NC, NS, C2, CH)
    dst2 = jnp.broadcast_to(dst.reshape(1, NS, C2, CH), (NC, NS, C2, CH))
    oh2 = jnp.zeros((2, CH, F), jnp.float32).at[0, :, 0].set(1.0).at[1, :, 1].set(1.0)
    zfeat = jnp.zeros((NP, F), jnp.float32)

    deg = _deg_call()(src1, dst1, oh2, zfeat)
    h0, ns, nd = _prep_call(deg, x_p)
    agg1 = _spmm1_call()(h0, src1, dst1, zfeat)
    g1 = _l1_call(agg1, nd, ns, W1, b1.reshape(1, H))
    tbl2 = g1.reshape(NC * NP, F)
    agg2 = _spmm2_call()(tbl2, src2, dst2, zfeat)
    wfc_p = jnp.zeros((H, 128), jnp.float32).at[:, :NCLS].set(Wfc)
    bfc_p = jnp.zeros((1, 128), jnp.float32).at[0, :NCLS].set(bfc)
    outp = _l2_call(agg2, nd, W2, b2.reshape(1, H), wfc_p, bfc_p)
    return outp[:, :NCLS]


# speedup vs baseline: 6.4718x; 1.5737x over previous
import jax
import jax.numpy as jnp
from jax import lax
from jax.experimental import pallas as pl
from jax.experimental.pallas import tpu as pltpu
from jax.experimental.pallas import tpu_sc as plsc

N = 10000
NP = 10240
E = 320000
F = 128
H = 256
NCLS = 18
NC = 2
NS = 16
CH = 80
ROWS_T = NP // NS
C1 = E // (NC * NS * CH)
C2 = E // (NS * CH)
BR = NP // 16

import functools


@functools.cache
def _mesh():
    return plsc.VectorSubcoreMesh(core_axis_name="c", subcore_axis_name="s",
                                  num_cores=NC, num_subcores=NS)


def _deg_body(src3, dst3, oh2, zfeat, deg, idxs_v, idxd_v, ohs_v, ohd_v, acc_sh):
    c = lax.axis_index("c")
    s = lax.axis_index("s")
    r0 = s * ROWS_T
    pltpu.sync_copy(zfeat.at[pl.ds(r0, ROWS_T)], acc_sh.at[pl.ds(r0, ROWS_T)])
    pltpu.sync_copy(oh2.at[0], ohs_v)
    pltpu.sync_copy(oh2.at[1], ohd_v)
    plsc.subcore_barrier()

    def seg(g, carry):
        pltpu.sync_copy(src3.at[c, s, g], idxs_v)
        pltpu.sync_copy(dst3.at[c, s, g], idxd_v)

        def chunk(j, carry2):
            pltpu.sync_copy(ohs_v, acc_sh.at[idxs_v.at[j]], add=True)
            pltpu.sync_copy(ohd_v, acc_sh.at[idxd_v.at[j]], add=True)
            return carry2

        lax.fori_loop(0, SEG, chunk, 0)
        return carry

    lax.fori_loop(0, C1 // SEG, seg, 0)
    plsc.subcore_barrier()
    pltpu.sync_copy(acc_sh.at[pl.ds(r0, ROWS_T)], deg.at[c, pl.ds(r0, ROWS_T)])


@functools.cache
def _deg_call():
    return pl.kernel(
        _deg_body,
        out_type=jax.ShapeDtypeStruct((NC, NP, F), jnp.float32),
        mesh=_mesh(),
        scratch_types=[
            pltpu.VMEM((SEG, CH), jnp.int32),
            pltpu.VMEM((SEG, CH), jnp.int32),
            pltpu.VMEM((CH, F), jnp.float32),
            pltpu.VMEM((CH, F), jnp.float32),
            pltpu.VMEM_SHARED((NP, F), jnp.float32),
        ],
    )


SEG = 25


def _make_spmm(nchunks, tbl_rows):
    nseg = nchunks // SEG

    def body(tbl, src3, dst3, zfeat, out, idxg_v, idxd_v, rows0, rows1, acc_sh,
             gs0, gs1):
        c = lax.axis_index("c")
        s = lax.axis_index("s")
        r0 = s * ROWS_T
        pltpu.sync_copy(zfeat.at[pl.ds(r0, ROWS_T)], acc_sh.at[pl.ds(r0, ROWS_T)])
        plsc.subcore_barrier()

        def seg(g, carry):
            pltpu.sync_copy(src3.at[c, s, g], idxg_v)
            pltpu.sync_copy(dst3.at[c, s, g], idxd_v)
            pltpu.async_copy(tbl.at[idxg_v.at[0]], rows0, gs0)

            def pair(i, carry2):
                j = 2 * i
                pltpu.make_async_copy(tbl.at[idxg_v.at[j]], rows0, gs0).wait()
                pltpu.async_copy(tbl.at[idxg_v.at[j + 1]], rows1, gs1)
                pltpu.sync_copy(rows0, acc_sh.at[idxd_v.at[j]], add=True)
                jn = jnp.where(j + 2 >= SEG, 0, j + 2)
                pltpu.make_async_copy(tbl.at[idxg_v.at[j + 1]], rows1, gs1).wait()
                pltpu.async_copy(tbl.at[idxg_v.at[jn]], rows0, gs0)
                pltpu.sync_copy(rows1, acc_sh.at[idxd_v.at[j + 1]], add=True)
                return carry2

            lax.fori_loop(0, SEG // 2, pair, 0)
            pltpu.make_async_copy(tbl.at[idxg_v.at[SEG - 1]], rows0, gs0).wait()
            pltpu.sync_copy(rows0, acc_sh.at[idxd_v.at[SEG - 1]], add=True)
            return carry

        lax.fori_loop(0, nseg, seg, 0)
        plsc.subcore_barrier()
        pltpu.sync_copy(acc_sh.at[pl.ds(r0, ROWS_T)], out.at[c, pl.ds(r0, ROWS_T)])

    return pl.kernel(
        body,
        out_type=jax.ShapeDtypeStruct((NC, NP, F), jnp.float32),
        mesh=_mesh(),
        scratch_types=[
            pltpu.VMEM((SEG, CH), jnp.int32),
            pltpu.VMEM((SEG, CH), jnp.int32),
            pltpu.VMEM((CH, F), jnp.float32),
            pltpu.VMEM((CH, F), jnp.float32),
            pltpu.VMEM_SHARED((NP, F), jnp.float32),
            pltpu.SemaphoreType.DMA,
            pltpu.SemaphoreType.DMA,
        ],
    )


_spmm1_call = functools.cache(lambda: _make_spmm(C1, NP))
_spmm2_call = functools.cache(lambda: _make_spmm(C2, NC * NP))


def _prep_body(deg_ref, x_ref, h0_ref, ns_ref, nd_ref):
    dsum = deg_ref[0] + deg_ref[1]
    ns = lax.rsqrt(jnp.maximum(dsum[:, 0:1], 1.0))
    nd = lax.rsqrt(jnp.maximum(dsum[:, 1:2], 1.0))
    h0_ref[...] = x_ref[...] * ns
    ns_ref[...] = ns
    nd_ref[...] = nd


def _prep_call(deg, x_p):
    return pl.pallas_call(
        _prep_body,
        grid=(NP // BR,),
        in_specs=[
            pl.BlockSpec((NC, BR, F), lambda i: (0, i, 0)),
            pl.BlockSpec((BR, F), lambda i: (i, 0)),
        ],
        out_specs=[
            pl.BlockSpec((BR, F), lambda i: (i, 0)),
            pl.BlockSpec((BR, 1), lambda i: (i, 0)),
            pl.BlockSpec((BR, 1), lambda i: (i, 0)),
        ],
        out_shape=[jax.ShapeDtypeStruct((NP, F), jnp.float32),
                   jax.ShapeDtypeStruct((NP, 1), jnp.float32),
                   jax.ShapeDtypeStruct((NP, 1), jnp.float32)],
    )(deg, x_p)


def _l1_body(a_ref, nd_ref, ns_ref, w_ref, b_ref, g_ref):
    agg = (a_ref[0] + a_ref[1]) * nd_ref[...]
    h = jnp.dot(agg, w_ref[...], preferred_element_type=jnp.float32,
                precision=lax.Precision.HIGHEST) + b_ref[...]
    h = jnp.maximum(h, 0.0) * ns_ref[...]
    g_ref[0] = h[:, :F]
    g_ref[1] = h[:, F:]


def _l1_call(agg1, nd, ns, W1, b1):
    return pl.pallas_call(
        _l1_body,
        grid=(NP // BR,),
        in_specs=[
            pl.BlockSpec((NC, BR, F), lambda i: (0, i, 0)),
            pl.BlockSpec((BR, 1), lambda i: (i, 0)),
            pl.BlockSpec((BR, 1), lambda i: (i, 0)),
            pl.BlockSpec((F, H), lambda i: (0, 0)),
            pl.BlockSpec((1, H), lambda i: (0, 0)),
        ],
        out_specs=pl.BlockSpec((NC, BR, F), lambda i: (0, i, 0)),
        out_shape=jax.ShapeDtypeStruct((NC, NP, F), jnp.float32),
    )(agg1, nd, ns, W1, b1)


def _l2_body(a_ref, nd_ref, w_ref, b_ref, wfc_ref, bfc_ref, o_ref, acc_ref):
    i = pl.program_id(0)

    @pl.when(i == 0)
    def _():
        acc_ref[...] = jnp.zeros_like(acc_ref)

    agg = jnp.concatenate([a_ref[0], a_ref[1]], axis=1) * nd_ref[...]
    h = jnp.dot(agg, w_ref[...], preferred_element_type=jnp.float32,
                precision=lax.Precision.HIGHEST) + b_ref[...]
    h = jnp.maximum(h, 0.0)
    rid = lax.broadcasted_iota(jnp.int32, h.shape, 0) + i * BR
    h = jnp.where(rid < N, h, 0.0)
    acc_ref[...] += jnp.sum(h, axis=0, keepdims=True)

    @pl.when(i == NP // BR - 1)
    def _():
        hg = acc_ref[...] * (1.0 / N)
        o_ref[...] = jnp.dot(hg, wfc_ref[...], preferred_element_type=jnp.float32,
                             precision=lax.Precision.HIGHEST) + bfc_ref[...]


def _l2_call(agg2, nd, W2, b2, wfc_p, bfc_p):
    return pl.pallas_call(
        _l2_body,
        grid=(NP // BR,),
        in_specs=[
            pl.BlockSpec((NC, BR, F), lambda i: (0, i, 0)),
            pl.BlockSpec((BR, 1), lambda i: (i, 0)),
            pl.BlockSpec((H, H), lambda i: (0, 0)),
            pl.BlockSpec((1, H), lambda i: (0, 0)),
            pl.BlockSpec((H, 128), lambda i: (0, 0)),
            pl.BlockSpec((1, 128), lambda i: (0, 0)),
        ],
        out_specs=pl.BlockSpec((1, 128), lambda i: (0, 0)),
        out_shape=jax.ShapeDtypeStruct((1, 128), jnp.float32),
        scratch_shapes=[pltpu.VMEM((1, H), jnp.float32)],
    )(agg2, nd, W2, b2, wfc_p, bfc_p)


def kernel(x, edge_index, W1, b1, W2, b2, Wfc, bfc):
    src = edge_index[0].astype(jnp.int32)
    dst = edge_index[1].astype(jnp.int32)
    x_p = jnp.zeros((NP, F), jnp.float32).at[:N].set(x)
    src1 = src.reshape(NC, NS, C1 // SEG, SEG, CH)
    dst1 = dst.reshape(NC, NS, C1 // SEG, SEG, CH)
    src2 = jnp.stack([src, src + NP]).reshape(NC, NS, C2 // SEG, SEG, CH)
    dst2 = jnp.broadcast_to(dst.reshape(1, NS, C2 // SEG, SEG, CH),
                            (NC, NS, C2 // SEG, SEG, CH))
    oh2 = jnp.zeros((2, CH, F), jnp.float32).at[0, :, 0].set(1.0).at[1, :, 1].set(1.0)
    zfeat = jnp.zeros((NP, F), jnp.float32)

    deg = _deg_call()(src1, dst1, oh2, zfeat)
    h0, ns, nd = _prep_call(deg, x_p)
    agg1 = _spmm1_call()(h0, src1, dst1, zfeat)
    g1 = _l1_call(agg1, nd, ns, W1, b1.reshape(1, H))
    tbl2 = g1.reshape(NC * NP, F)
    agg2 = _spmm2_call()(tbl2, src2, dst2, zfeat)
    wfc_p = jnp.zeros((H, 128), jnp.float32).at[:, :NCLS].set(Wfc)
    bfc_p = jnp.zeros((1, 128), jnp.float32).at[0, :NCLS].set(bfc)
    outp = _l2_call(agg2, nd, W2, b2.reshape(1, H), wfc_p, bfc_p)
    return outp[:, :NCLS]


# speedup vs baseline: 8.2603x; 1.2764x over previous
import jax
import jax.numpy as jnp
from jax import lax
from jax.experimental import pallas as pl
from jax.experimental.pallas import tpu as pltpu
from jax.experimental.pallas import tpu_sc as plsc

N = 10000
NP = 10240
E = 320000
F = 128
H = 256
NCLS = 18
NC = 2
NS = 16
CH = 80
ROWS_T = NP // NS
C1 = E // (NC * NS * CH)
C2 = E // (NS * CH)
BR = NP // 16

import functools


@functools.cache
def _mesh():
    return plsc.VectorSubcoreMesh(core_axis_name="c", subcore_axis_name="s",
                                  num_cores=NC, num_subcores=NS)


def _deg_body(src3, dst3, oh2, zfeat, deg, idxs_v, idxd_v, ohs_v, ohd_v, acc_sh,
              ssem):
    c = lax.axis_index("c")
    s = lax.axis_index("s")
    r0 = s * ROWS_T
    pltpu.sync_copy(zfeat.at[pl.ds(r0, ROWS_T)], acc_sh.at[pl.ds(r0, ROWS_T)])
    pltpu.sync_copy(oh2.at[0], ohs_v)
    pltpu.sync_copy(oh2.at[1], ohd_v)
    plsc.subcore_barrier()

    def seg(g, carry):
        pltpu.sync_copy(src3.at[c, s, g], idxs_v)
        pltpu.sync_copy(dst3.at[c, s, g], idxd_v)

        def chunk(j, carry2):
            pltpu.async_copy(ohs_v, acc_sh.at[idxs_v.at[j]], ssem, add=True)
            pltpu.async_copy(ohd_v, acc_sh.at[idxd_v.at[j]], ssem, add=True)
            return carry2

        lax.fori_loop(0, SEG, chunk, 0)

        def drain(j, carry2):
            pltpu.make_async_copy(ohs_v, acc_sh.at[idxs_v.at[j]], ssem).wait()
            pltpu.make_async_copy(ohd_v, acc_sh.at[idxd_v.at[j]], ssem).wait()
            return carry2

        lax.fori_loop(0, SEG, drain, 0)
        return carry

    lax.fori_loop(0, C1 // SEG, seg, 0)
    plsc.subcore_barrier()
    pltpu.sync_copy(acc_sh.at[pl.ds(r0, ROWS_T)], deg.at[c, pl.ds(r0, ROWS_T)])


@functools.cache
def _deg_call():
    return pl.kernel(
        _deg_body,
        out_type=jax.ShapeDtypeStruct((NC, NP, F), jnp.float32),
        mesh=_mesh(),
        scratch_types=[
            pltpu.VMEM((SEG, CH), jnp.int32),
            pltpu.VMEM((SEG, CH), jnp.int32),
            pltpu.VMEM((CH, F), jnp.float32),
            pltpu.VMEM((CH, F), jnp.float32),
            pltpu.VMEM_SHARED((NP, F), jnp.float32),
            pltpu.SemaphoreType.DMA,
        ],
    )


SEG = 25


def _make_spmm(nchunks, tbl_rows):
    nseg = nchunks // SEG

    def body(tbl, src3, dst3, zfeat, out, idxg_v, idxd_v, rows0, rows1, rows2,
             acc_sh, gs0, gs1, gs2, ss0, ss1, ss2):
        c = lax.axis_index("c")
        s = lax.axis_index("s")
        r0 = s * ROWS_T
        pltpu.sync_copy(zfeat.at[pl.ds(r0, ROWS_T)], acc_sh.at[pl.ds(r0, ROWS_T)])
        plsc.subcore_barrier()

        rows = (rows0, rows1, rows2)
        gss = (gs0, gs1, gs2)
        sss = (ss0, ss1, ss2)

        def gather(j, b, gsem):
            pltpu.async_copy(tbl.at[idxg_v.at[j]], rows[b], gss[gsem])

        def seg(g, carry):
            pltpu.sync_copy(src3.at[c, s, g], idxg_v)
            pltpu.sync_copy(dst3.at[c, s, g], idxd_v)
            gather(0, 0, 0)
            gather(1, 1, 1)
            gather(2, 2, 2)
            pltpu.make_async_copy(tbl.at[idxg_v.at[0]], rows0, gs0).wait()
            pltpu.async_copy(rows0, acc_sh.at[idxd_v.at[0]], ss0, add=True)

            def triple(i, carry2):
                j = 3 * i

                def step(jj, b):
                    bp = (b + 2) % 3
                    pltpu.make_async_copy(rows[bp],
                                          acc_sh.at[idxd_v.at[jj - 1]],
                                          sss[bp]).wait()
                    jn = jnp.where(jj + 2 >= SEG, 0, jj + 2)
                    gather(jn, bp, bp)
                    pltpu.make_async_copy(tbl.at[idxg_v.at[jj]], rows[b],
                                          gss[b]).wait()
                    pltpu.async_copy(rows[b], acc_sh.at[idxd_v.at[jj]],
                                     sss[b], add=True)

                step(j + 1, 1)
                step(j + 2, 2)
                step(j + 3, 0)
                return carry2

            lax.fori_loop(0, (SEG - 1) // 3, triple, 0)
            pltpu.make_async_copy(rows0, acc_sh.at[idxd_v.at[SEG - 1]],
                                  ss0).wait()
            pltpu.make_async_copy(tbl.at[idxg_v.at[0]], rows1, gs1).wait()
            pltpu.make_async_copy(tbl.at[idxg_v.at[0]], rows2, gs2).wait()
            return carry

        lax.fori_loop(0, nseg, seg, 0)
        plsc.subcore_barrier()
        pltpu.sync_copy(acc_sh.at[pl.ds(r0, ROWS_T)], out.at[c, pl.ds(r0, ROWS_T)])

    return pl.kernel(
        body,
        out_type=jax.ShapeDtypeStruct((NC, NP, F), jnp.float32),
        mesh=_mesh(),
        scratch_types=[
            pltpu.VMEM((SEG, CH), jnp.int32),
            pltpu.VMEM((SEG, CH), jnp.int32),
            pltpu.VMEM((CH, F), jnp.float32),
            pltpu.VMEM((CH, F), jnp.float32),
            pltpu.VMEM((CH, F), jnp.float32),
            pltpu.VMEM_SHARED((NP, F), jnp.float32),
            pltpu.SemaphoreType.DMA,
            pltpu.SemaphoreType.DMA,
            pltpu.SemaphoreType.DMA,
            pltpu.SemaphoreType.DMA,
            pltpu.SemaphoreType.DMA,
            pltpu.SemaphoreType.DMA,
        ],
    )


_spmm1_call = functools.cache(lambda: _make_spmm(C1, NP))
_spmm2_call = functools.cache(lambda: _make_spmm(C2, NC * NP))


def _prep_body(deg_ref, x_ref, h0_ref, ns_ref, nd_ref):
    dsum = deg_ref[0] + deg_ref[1]
    ns = lax.rsqrt(jnp.maximum(dsum[:, 0:1], 1.0))
    nd = lax.rsqrt(jnp.maximum(dsum[:, 1:2], 1.0))
    h0_ref[...] = x_ref[...] * ns
    ns_ref[...] = ns
    nd_ref[...] = nd


def _prep_call(deg, x_p):
    return pl.pallas_call(
        _prep_body,
        grid=(NP // BR,),
        in_specs=[
            pl.BlockSpec((NC, BR, F), lambda i: (0, i, 0)),
            pl.BlockSpec((BR, F), lambda i: (i, 0)),
        ],
        out_specs=[
            pl.BlockSpec((BR, F), lambda i: (i, 0)),
            pl.BlockSpec((BR, 1), lambda i: (i, 0)),
            pl.BlockSpec((BR, 1), lambda i: (i, 0)),
        ],
        out_shape=[jax.ShapeDtypeStruct((NP, F), jnp.float32),
                   jax.ShapeDtypeStruct((NP, 1), jnp.float32),
                   jax.ShapeDtypeStruct((NP, 1), jnp.float32)],
    )(deg, x_p)


def _l1_body(a_ref, nd_ref, ns_ref, w_ref, b_ref, g_ref):
    agg = (a_ref[0] + a_ref[1]) * nd_ref[...]
    h = jnp.dot(agg, w_ref[...], preferred_element_type=jnp.float32,
                precision=lax.Precision.HIGHEST) + b_ref[...]
    h = jnp.maximum(h, 0.0) * ns_ref[...]
    g_ref[0] = h[:, :F]
    g_ref[1] = h[:, F:]


def _l1_call(agg1, nd, ns, W1, b1):
    return pl.pallas_call(
        _l1_body,
        grid=(NP // BR,),
        in_specs=[
            pl.BlockSpec((NC, BR, F), lambda i: (0, i, 0)),
            pl.BlockSpec((BR, 1), lambda i: (i, 0)),
            pl.BlockSpec((BR, 1), lambda i: (i, 0)),
            pl.BlockSpec((F, H), lambda i: (0, 0)),
            pl.BlockSpec((1, H), lambda i: (0, 0)),
        ],
        out_specs=pl.BlockSpec((NC, BR, F), lambda i: (0, i, 0)),
        out_shape=jax.ShapeDtypeStruct((NC, NP, F), jnp.float32),
    )(agg1, nd, ns, W1, b1)


def _l2_body(a_ref, nd_ref, w_ref, b_ref, wfc_ref, bfc_ref, o_ref, acc_ref):
    i = pl.program_id(0)

    @pl.when(i == 0)
    def _():
        acc_ref[...] = jnp.zeros_like(acc_ref)

    agg = jnp.concatenate([a_ref[0], a_ref[1]], axis=1) * nd_ref[...]
    h = jnp.dot(agg, w_ref[...], preferred_element_type=jnp.float32,
                precision=lax.Precision.HIGHEST) + b_ref[...]
    h = jnp.maximum(h, 0.0)
    rid = lax.broadcasted_iota(jnp.int32, h.shape, 0) + i * BR
    h = jnp.where(rid < N, h, 0.0)
    acc_ref[...] += jnp.sum(h, axis=0, keepdims=True)

    @pl.when(i == NP // BR - 1)
    def _():
        hg = acc_ref[...] * (1.0 / N)
        o_ref[...] = jnp.dot(hg, wfc_ref[...], preferred_element_type=jnp.float32,
                             precision=lax.Precision.HIGHEST) + bfc_ref[...]


def _l2_call(agg2, nd, W2, b2, wfc_p, bfc_p):
    return pl.pallas_call(
        _l2_body,
        grid=(NP // BR,),
        in_specs=[
            pl.BlockSpec((NC, BR, F), lambda i: (0, i, 0)),
            pl.BlockSpec((BR, 1), lambda i: (i, 0)),
            pl.BlockSpec((H, H), lambda i: (0, 0)),
            pl.BlockSpec((1, H), lambda i: (0, 0)),
            pl.BlockSpec((H, 128), lambda i: (0, 0)),
            pl.BlockSpec((1, 128), lambda i: (0, 0)),
        ],
        out_specs=pl.BlockSpec((1, 128), lambda i: (0, 0)),
        out_shape=jax.ShapeDtypeStruct((1, 128), jnp.float32),
        scratch_shapes=[pltpu.VMEM((1, H), jnp.float32)],
    )(agg2, nd, W2, b2, wfc_p, bfc_p)


def kernel(x, edge_index, W1, b1, W2, b2, Wfc, bfc):
    src = edge_index[0].astype(jnp.int32)
    dst = edge_index[1].astype(jnp.int32)
    x_p = jnp.zeros((NP, F), jnp.float32).at[:N].set(x)
    src1 = src.reshape(NC, NS, C1 // SEG, SEG, CH)
    dst1 = dst.reshape(NC, NS, C1 // SEG, SEG, CH)
    src2 = jnp.stack([src, src + NP]).reshape(NC, NS, C2 // SEG, SEG, CH)
    dst2 = jnp.broadcast_to(dst.reshape(1, NS, C2 // SEG, SEG, CH),
                            (NC, NS, C2 // SEG, SEG, CH))
    oh2 = jnp.zeros((2, CH, F), jnp.float32).at[0, :, 0].set(1.0).at[1, :, 1].set(1.0)
    zfeat = jnp.zeros((NP, F), jnp.float32)

    deg = _deg_call()(src1, dst1, oh2, zfeat)
    h0, ns, nd = _prep_call(deg, x_p)
    agg1 = _spmm1_call()(h0, src1, dst1, zfeat)
    g1 = _l1_call(agg1, nd, ns, W1, b1.reshape(1, H))
    tbl2 = g1.reshape(NC * NP, F)
    agg2 = _spmm2_call()(tbl2, src2, dst2, zfeat)
    wfc_p = jnp.zeros((H, 128), jnp.float32).at[:, :NCLS].set(Wfc)
    bfc_p = jnp.zeros((1, 128), jnp.float32).at[0, :NCLS].set(bfc)
    outp = _l2_call(agg2, nd, W2, b2.reshape(1, H), wfc_p, bfc_p)
    return outp[:, :NCLS]


# speedup vs baseline: 8.3451x; 1.0103x over previous
import jax
import jax.numpy as jnp
from jax import lax
from jax.experimental import pallas as pl
from jax.experimental.pallas import tpu as pltpu
from jax.experimental.pallas import tpu_sc as plsc

N = 10000
NP = 10240
E = 320000
F = 128
H = 256
NCLS = 18
NC = 2
NS = 16
CH = 80
ROWS_T = NP // NS
C1 = E // (NC * NS * CH)
C2 = E // (NS * CH)
BR = NP // 16

import functools


@functools.cache
def _mesh():
    return plsc.VectorSubcoreMesh(core_axis_name="c", subcore_axis_name="s",
                                  num_cores=NC, num_subcores=NS)


def _deg_body(src3, dst3, oh2, zfeat, deg, idxs_v, idxd_v, ohs_v, ohd_v, acc_sh,
              ssem):
    c = lax.axis_index("c")
    s = lax.axis_index("s")
    r0 = s * ROWS_T
    pltpu.sync_copy(zfeat, acc_sh.at[pl.ds(r0, ROWS_T)])
    pltpu.sync_copy(oh2.at[0], ohs_v)
    pltpu.sync_copy(oh2.at[1], ohd_v)
    plsc.subcore_barrier()

    def seg(g, carry):
        pltpu.sync_copy(src3.at[c, s, g], idxs_v)
        pltpu.sync_copy(dst3.at[c, s, g], idxd_v)

        def chunk(j, carry2):
            pltpu.async_copy(ohs_v, acc_sh.at[idxs_v.at[j]], ssem, add=True)
            pltpu.async_copy(ohd_v, acc_sh.at[idxd_v.at[j]], ssem, add=True)
            return carry2

        lax.fori_loop(0, SEG, chunk, 0)

        def drain(j, carry2):
            pltpu.make_async_copy(ohs_v, acc_sh.at[idxs_v.at[j]], ssem).wait()
            pltpu.make_async_copy(ohd_v, acc_sh.at[idxd_v.at[j]], ssem).wait()
            return carry2

        lax.fori_loop(0, SEG, drain, 0)
        return carry

    lax.fori_loop(0, C1 // SEG, seg, 0)
    plsc.subcore_barrier()
    pltpu.sync_copy(acc_sh.at[pl.ds(r0, ROWS_T)], deg.at[c, pl.ds(r0, ROWS_T)])


@functools.cache
def _deg_call():
    return pl.kernel(
        _deg_body,
        out_type=jax.ShapeDtypeStruct((NC, NP, F), jnp.float32),
        mesh=_mesh(),
        scratch_types=[
            pltpu.VMEM((SEG, CH), jnp.int32),
            pltpu.VMEM((SEG, CH), jnp.int32),
            pltpu.VMEM((CH, F), jnp.float32),
            pltpu.VMEM((CH, F), jnp.float32),
            pltpu.VMEM_SHARED((NP, F), jnp.float32),
            pltpu.SemaphoreType.DMA,
        ],
    )


SEG = 25


def _make_spmm(nchunks, tbl_rows):
    nseg = nchunks // SEG

    def body(tbl, src3, dst3, zfeat, out, idxg_v, idxd_v, rows0, rows1, rows2,
             acc_sh, gs0, gs1, gs2, ss0, ss1, ss2):
        c = lax.axis_index("c")
        s = lax.axis_index("s")
        r0 = s * ROWS_T
        pltpu.sync_copy(zfeat, acc_sh.at[pl.ds(r0, ROWS_T)])
        plsc.subcore_barrier()

        rows = (rows0, rows1, rows2)
        gss = (gs0, gs1, gs2)
        sss = (ss0, ss1, ss2)

        def gather(j, b, gsem):
            pltpu.async_copy(tbl.at[idxg_v.at[j]], rows[b], gss[gsem])

        def seg(g, carry):
            pltpu.sync_copy(src3.at[c, s, g], idxg_v)
            pltpu.sync_copy(dst3.at[c, s, g], idxd_v)
            gather(0, 0, 0)
            gather(1, 1, 1)
            gather(2, 2, 2)
            pltpu.make_async_copy(tbl.at[idxg_v.at[0]], rows0, gs0).wait()
            pltpu.async_copy(rows0, acc_sh.at[idxd_v.at[0]], ss0, add=True)

            def triple(i, carry2):
                j = 3 * i

                def step(jj, b):
                    bp = (b + 2) % 3
                    pltpu.make_async_copy(rows[bp],
                                          acc_sh.at[idxd_v.at[jj - 1]],
                                          sss[bp]).wait()
                    jn = jnp.where(jj + 2 >= SEG, 0, jj + 2)
                    gather(jn, bp, bp)
                    pltpu.make_async_copy(tbl.at[idxg_v.at[jj]], rows[b],
                                          gss[b]).wait()
                    pltpu.async_copy(rows[b], acc_sh.at[idxd_v.at[jj]],
                                     sss[b], add=True)

                step(j + 1, 1)
                step(j + 2, 2)
                step(j + 3, 0)
                return carry2

            lax.fori_loop(0, (SEG - 1) // 3, triple, 0)
            pltpu.make_async_copy(rows0, acc_sh.at[idxd_v.at[SEG - 1]],
                                  ss0).wait()
            pltpu.make_async_copy(tbl.at[idxg_v.at[0]], rows1, gs1).wait()
            pltpu.make_async_copy(tbl.at[idxg_v.at[0]], rows2, gs2).wait()
            return carry

        lax.fori_loop(0, nseg, seg, 0)
        plsc.subcore_barrier()
        pltpu.sync_copy(acc_sh.at[pl.ds(r0, ROWS_T)], out.at[c, pl.ds(r0, ROWS_T)])

    return pl.kernel(
        body,
        out_type=jax.ShapeDtypeStruct((NC, NP, F), jnp.float32),
        mesh=_mesh(),
        scratch_types=[
            pltpu.VMEM((SEG, CH), jnp.int32),
            pltpu.VMEM((SEG, CH), jnp.int32),
            pltpu.VMEM((CH, F), jnp.float32),
            pltpu.VMEM((CH, F), jnp.float32),
            pltpu.VMEM((CH, F), jnp.float32),
            pltpu.VMEM_SHARED((NP, F), jnp.float32),
            pltpu.SemaphoreType.DMA,
            pltpu.SemaphoreType.DMA,
            pltpu.SemaphoreType.DMA,
            pltpu.SemaphoreType.DMA,
            pltpu.SemaphoreType.DMA,
            pltpu.SemaphoreType.DMA,
        ],
    )


_spmm1_call = functools.cache(lambda: _make_spmm(C1, NP))
_spmm2_call = functools.cache(lambda: _make_spmm(C2, NC * NP))


def _prep_body(deg_ref, x_ref, h0_ref, ns_ref, nd_ref):
    dsum = deg_ref[0] + deg_ref[1]
    ns = lax.rsqrt(jnp.maximum(dsum[:, 0:1], 1.0))
    nd = lax.rsqrt(jnp.maximum(dsum[:, 1:2], 1.0))
    h0_ref[...] = x_ref[...] * ns
    ns_ref[...] = ns
    nd_ref[...] = nd


def _prep_call(deg, x_p):
    return pl.pallas_call(
        _prep_body,
        grid=(NP // BR,),
        in_specs=[
            pl.BlockSpec((NC, BR, F), lambda i: (0, i, 0)),
            pl.BlockSpec((BR, F), lambda i: (i, 0)),
        ],
        out_specs=[
            pl.BlockSpec((BR, F), lambda i: (i, 0)),
            pl.BlockSpec((BR, 1), lambda i: (i, 0)),
            pl.BlockSpec((BR, 1), lambda i: (i, 0)),
        ],
        out_shape=[jax.ShapeDtypeStruct((NP, F), jnp.float32),
                   jax.ShapeDtypeStruct((NP, 1), jnp.float32),
                   jax.ShapeDtypeStruct((NP, 1), jnp.float32)],
    )(deg, x_p)


def _l1_body(a_ref, nd_ref, ns_ref, w_ref, b_ref, g_ref):
    agg = (a_ref[0] + a_ref[1]) * nd_ref[...]
    h = jnp.dot(agg, w_ref[...], preferred_element_type=jnp.float32) + b_ref[...]
    h = jnp.maximum(h, 0.0) * ns_ref[...]
    g_ref[0] = h[:, :F]
    g_ref[1] = h[:, F:]


def _l1_call(agg1, nd, ns, W1, b1):
    return pl.pallas_call(
        _l1_body,
        grid=(NP // BR,),
        in_specs=[
            pl.BlockSpec((NC, BR, F), lambda i: (0, i, 0)),
            pl.BlockSpec((BR, 1), lambda i: (i, 0)),
            pl.BlockSpec((BR, 1), lambda i: (i, 0)),
            pl.BlockSpec((F, H), lambda i: (0, 0)),
            pl.BlockSpec((1, H), lambda i: (0, 0)),
        ],
        out_specs=pl.BlockSpec((NC, BR, F), lambda i: (0, i, 0)),
        out_shape=jax.ShapeDtypeStruct((NC, NP, F), jnp.float32),
    )(agg1, nd, ns, W1, b1)


def _l2_body(a_ref, nd_ref, w_ref, b_ref, wfc_ref, bfc_ref, o_ref, acc_ref):
    i = pl.program_id(0)

    @pl.when(i == 0)
    def _():
        acc_ref[...] = jnp.zeros_like(acc_ref)

    agg = jnp.concatenate([a_ref[0], a_ref[1]], axis=1) * nd_ref[...]
    h = jnp.dot(agg, w_ref[...], preferred_element_type=jnp.float32) + b_ref[...]
    h = jnp.maximum(h, 0.0)
    rid = lax.broadcasted_iota(jnp.int32, h.shape, 0) + i * BR
    h = jnp.where(rid < N, h, 0.0)
    acc_ref[...] += jnp.sum(h, axis=0, keepdims=True)

    @pl.when(i == NP // BR - 1)
    def _():
        hg = acc_ref[...] * (1.0 / N)
        o_ref[...] = jnp.dot(hg, wfc_ref[...], preferred_element_type=jnp.float32) + bfc_ref[...]


def _l2_call(agg2, nd, W2, b2, wfc_p, bfc_p):
    return pl.pallas_call(
        _l2_body,
        grid=(NP // BR,),
        in_specs=[
            pl.BlockSpec((NC, BR, F), lambda i: (0, i, 0)),
            pl.BlockSpec((BR, 1), lambda i: (i, 0)),
            pl.BlockSpec((H, H), lambda i: (0, 0)),
            pl.BlockSpec((1, H), lambda i: (0, 0)),
            pl.BlockSpec((H, 128), lambda i: (0, 0)),
            pl.BlockSpec((1, 128), lambda i: (0, 0)),
        ],
        out_specs=pl.BlockSpec((1, 128), lambda i: (0, 0)),
        out_shape=jax.ShapeDtypeStruct((1, 128), jnp.float32),
        scratch_shapes=[pltpu.VMEM((1, H), jnp.float32)],
    )(agg2, nd, W2, b2, wfc_p, bfc_p)


def kernel(x, edge_index, W1, b1, W2, b2, Wfc, bfc):
    src = edge_index[0].astype(jnp.int32)
    dst = edge_index[1].astype(jnp.int32)
    x_p = jnp.zeros((NP, F), jnp.float32).at[:N].set(x)
    src1 = src.reshape(NC, NS, C1 // SEG, SEG, CH)
    dst1 = dst.reshape(NC, NS, C1 // SEG, SEG, CH)
    src2 = jnp.stack([src, src + NP]).reshape(NC, NS, C2 // SEG, SEG, CH)
    dst2 = jnp.broadcast_to(dst.reshape(1, NS, C2 // SEG, SEG, CH),
                            (NC, NS, C2 // SEG, SEG, CH))
    oh2 = jnp.zeros((2, CH, F), jnp.float32).at[0, :, 0].set(1.0).at[1, :, 1].set(1.0)
    zfeat = jnp.zeros((ROWS_T, F), jnp.float32)

    deg = _deg_call()(src1, dst1, oh2, zfeat)
    h0, ns, nd = _prep_call(deg, x_p)
    agg1 = _spmm1_call()(h0, src1, dst1, zfeat)
    g1 = _l1_call(agg1, nd, ns, W1, b1.reshape(1, H))
    tbl2 = g1.reshape(NC * NP, F)
    agg2 = _spmm2_call()(tbl2, src2, dst2, zfeat)
    wfc_p = jnp.zeros((H, 128), jnp.float32).at[:, :NCLS].set(Wfc)
    bfc_p = jnp.zeros((1, 128), jnp.float32).at[0, :NCLS].set(bfc)
    outp = _l2_call(agg2, nd, W2, b2.reshape(1, H), wfc_p, bfc_p)
    return outp[:, :NCLS]
